# baseline (speedup 1.0000x reference)
import functools
import math

import jax
import jax.numpy as jnp
from jax import lax
from jax.experimental import pallas as pl
from jax.experimental.pallas import tpu as pltpu
from jax.experimental.pallas import tpu_sc as plsc

_NC = 2
_NS = 16
_NW = _NC * _NS
_LANES = 16
_CHUNK = 128
_NBUF = 4


@functools.lru_cache(maxsize=None)
def _make_lookup(B, V, D, T, scale):
    N = B // T
    assert D % _LANES == 0 and N % _CHUNK == 0 and D % 8 == 0
    b_per_w = B // _NW
    assert b_per_w % (_CHUNK * _NBUF) == 0
    n_chunks = b_per_w // _CHUNK
    n_outer = n_chunks // _NBUF
    jcols = N // _CHUNK
    piece = 8 * _CHUNK
    npiece = D // 8
    mesh = plsc.VectorSubcoreMesh(core_axis_name="c", subcore_axis_name="s")

    @functools.partial(
        pl.kernel,
        mesh=mesh,
        out_type=jax.ShapeDtypeStruct((B * D,), jnp.float32),
        scratch_types=(
            [pltpu.VMEM((b_per_w,), jnp.int32)]
            + [pltpu.VMEM((_CHUNK, D), jnp.float32) for _ in range(_NBUF)]
            + [pltpu.VMEM((_CHUNK * D,), jnp.float32) for _ in range(_NBUF)]
            + [pltpu.SemaphoreType.DMA for _ in range(2 * _NBUF)]
        ),
        compiler_params=pltpu.CompilerParams(
            use_tc_tiling_on_sc=False, needs_layout_passes=False
        ),
    )
    def lookup(idx_hbm, table_hbm, out_hbm, idx_v, *rest):
        g_buf = rest[:_NBUF]
        t_buf = rest[_NBUF:2 * _NBUF]
        sem_g = rest[2 * _NBUF:3 * _NBUF]
        sem_o = rest[3 * _NBUF:]

        wid = lax.axis_index("s") * _NC + lax.axis_index("c")
        base = wid * b_per_w
        c0 = wid * n_chunks
        pltpu.sync_copy(idx_hbm.at[pl.ds(base, b_per_w)], idx_v)

        def start_gather(b, c):
            start = pl.multiple_of(c * _CHUNK, _CHUNK)
            pltpu.async_copy(
                table_hbm.at[idx_v.at[pl.ds(start, _CHUNK)]], g_buf[b], sem_g[b]
            )

        for b in range(_NBUF):
            start_gather(b, b)

        lane = lax.iota(jnp.int32, _LANES)
        tbase = [(lane + k * _LANES) * _CHUNK for k in range(D // _LANES)]

        def outer(g, carry):
            for b in range(_NBUF):
                c = g * _NBUF + b
                pltpu.make_async_copy(
                    table_hbm.at[pl.ds(0, _CHUNK)], g_buf[b], sem_g[b]
                ).wait()

                @plsc.parallel_loop(0, _CHUNK, step=1, unroll=8)
                def _(r, b=b):
                    for k in range(D // _LANES):
                        v = g_buf[b][r, pl.ds(k * _LANES, _LANES)]
                        plsc.store_scatter(t_buf[b], [tbase[k] + r], v * scale)

                @pl.when(g > 0)
                def _(b=b):
                    pltpu.make_async_copy(
                        out_hbm.at[pl.ds(0, _CHUNK * D)], t_buf[b], sem_o[b]
                    ).wait()

                cg = c0 + c
                t2 = cg // jcols
                j = cg % jcols
                for i in range(npiece):
                    qoff = ((t2 * npiece + i) * jcols + j) * piece
                    pltpu.async_copy(
                        t_buf[b].at[pl.ds(i * piece, piece)],
                        out_hbm.at[pl.ds(qoff, piece)],
                        sem_o[b],
                    )

                @pl.when(c + _NBUF < n_chunks)
                def _(b=b, c=c):
                    start_gather(b, c + _NBUF)
            return carry

        lax.fori_loop(0, n_outer, outer, 0)

        for b in range(_NBUF):
            pltpu.make_async_copy(
                out_hbm.at[pl.ds(0, _CHUNK * D)], t_buf[b], sem_o[b]
            ).wait()

    return lookup


def kernel(tokens, table):
    n, t = tokens.shape
    V, D = table.shape
    B = n * t
    idx = tokens.T.reshape(B).astype(jnp.int32)
    q = _make_lookup(B, V, D, t, float(math.sqrt(D)))(idx, table)
    q5 = q.reshape(t, D // 8, n // 128, 8, 128)
    return q5.transpose(2, 4, 0, 1, 3).reshape(n, t, D)


# speedup vs baseline: 1.7385x; 1.7385x over previous
import functools
import math

import jax
import jax.numpy as jnp
from jax import lax
from jax.experimental import pallas as pl
from jax.experimental.pallas import tpu as pltpu
from jax.experimental.pallas import tpu_sc as plsc

_NC = 2
_NS = 16
_NW = _NC * _NS
_LANES = 16
_CHUNK = 128
_NBUF = 4
_TPAD = _CHUNK + 1


@functools.lru_cache(maxsize=None)
def _make_lookup(B, V, D, T, scale):
    N = B // T
    assert D % _LANES == 0 and N % _CHUNK == 0 and D % 8 == 0
    b_per_w = B // _NW
    assert b_per_w % (_CHUNK * _NBUF) == 0
    n_chunks = b_per_w // _CHUNK
    n_outer = n_chunks // _NBUF
    jcols = N // _CHUNK
    npiece = D // 8
    mesh = plsc.VectorSubcoreMesh(core_axis_name="c", subcore_axis_name="s")

    @functools.partial(
        pl.kernel,
        mesh=mesh,
        out_type=jax.ShapeDtypeStruct((B * D // _CHUNK, _CHUNK), jnp.float32),
        scratch_types=(
            [pltpu.VMEM((b_per_w,), jnp.int32)]
            + [pltpu.VMEM((_CHUNK, D), jnp.float32) for _ in range(_NBUF)]
            + [pltpu.VMEM((D, _TPAD), jnp.float32) for _ in range(_NBUF)]
            + [pltpu.SemaphoreType.DMA for _ in range(2 * _NBUF)]
        ),
        compiler_params=pltpu.CompilerParams(
            use_tc_tiling_on_sc=False, needs_layout_passes=False
        ),
    )
    def lookup(idx_hbm, table_hbm, out_hbm, idx_v, *rest):
        g_buf = rest[:_NBUF]
        t_buf = rest[_NBUF:2 * _NBUF]
        sem_g = rest[2 * _NBUF:3 * _NBUF]
        sem_o = rest[3 * _NBUF:]

        wid = lax.axis_index("s") * _NC + lax.axis_index("c")
        base = wid * b_per_w
        c0 = wid * n_chunks
        pltpu.sync_copy(idx_hbm.at[pl.ds(base, b_per_w)], idx_v)

        def start_gather(b, c):
            start = pl.multiple_of(c * _CHUNK, _CHUNK)
            pltpu.async_copy(
                table_hbm.at[idx_v.at[pl.ds(start, _CHUNK)]], g_buf[b], sem_g[b]
            )

        for b in range(_NBUF):
            start_gather(b, b)

        lane = lax.iota(jnp.int32, _LANES)
        cvecs = [lane + k * _LANES for k in range(D // _LANES)]
        zero = lane * 0

        def outer(g, carry):
            for b in range(_NBUF):
                c = g * _NBUF + b
                pltpu.make_async_copy(
                    table_hbm.at[pl.ds(0, _CHUNK)], g_buf[b], sem_g[b]
                ).wait()

                @plsc.parallel_loop(0, _CHUNK, step=1, unroll=8)
                def _(r, b=b):
                    rvec = zero + r
                    for k in range(D // _LANES):
                        v = g_buf[b][r, pl.ds(k * _LANES, _LANES)]
                        plsc.store_scatter(t_buf[b], [cvecs[k], rvec], v * scale)

                @pl.when(g > 0)
                def _(b=b):
                    pltpu.make_async_copy(
                        out_hbm.at[pl.ds(0, D)],
                        t_buf[b].at[pl.ds(0, D), pl.ds(0, _CHUNK)],
                        sem_o[b],
                    ).wait()

                cg = c0 + c
                t2 = cg // jcols
                j = cg % jcols
                for i in range(npiece):
                    qrow = ((t2 * npiece + i) * jcols + j) * 8
                    pltpu.async_copy(
                        t_buf[b].at[pl.ds(i * 8, 8), pl.ds(0, _CHUNK)],
                        out_hbm.at[pl.ds(qrow, 8)],
                        sem_o[b],
                    )

                @pl.when(c + _NBUF < n_chunks)
                def _(b=b, c=c):
                    start_gather(b, c + _NBUF)
            return carry

        lax.fori_loop(0, n_outer, outer, 0)

        for b in range(_NBUF):
            pltpu.make_async_copy(
                out_hbm.at[pl.ds(0, D)],
                t_buf[b].at[pl.ds(0, D), pl.ds(0, _CHUNK)],
                sem_o[b],
            ).wait()

    return lookup


def kernel(tokens, table):
    n, t = tokens.shape
    V, D = table.shape
    B = n * t
    idx = tokens.T.reshape(B).astype(jnp.int32)
    q = _make_lookup(B, V, D, t, float(math.sqrt(D)))(idx, table)
    q5 = q.reshape(t, D // 8, n // 128, 8, 128)
    return q5.transpose(2, 4, 0, 1, 3).reshape(n, t, D)
